# baseline (speedup 1.0000x reference)
import jax
import jax.numpy as jnp
from jax.experimental import pallas as pl
from jax.experimental.pallas import tpu as pltpu

B, N, D, H, R = 4096, 200, 16, 16, 8
BBLK = 64
TB = BBLK * N
GRID = B // BBLK


def _leaky(x):
    return jnp.maximum(x, 0.01 * x)


def _block_kernel(u_ref, i_ref, a_ref, o_ref, s_ref,
                  wui1a_ref, wui1b_ref, bui1_ref, wui2_ref, bui2_ref,
                  w1a_ref, w1b_ref, b1_ref, w2_ref, b2_ref, gt_ref,
                  out_ref):
    f32 = jnp.float32
    h_ui = _leaky(jnp.dot(u_ref[...], wui1a_ref[...], preferred_element_type=f32)
                  + jnp.dot(i_ref[...], wui1b_ref[...], preferred_element_type=f32)
                  + bui1_ref[...])
    ui_emb = _leaky(jnp.dot(h_ui, wui2_ref[...], preferred_element_type=f32)
                    + bui2_ref[...])
    ui_t = jnp.concatenate([ui_emb] * R, axis=-1)

    h_all = _leaky(jnp.dot(a_ref[...], w1a_ref[...], preferred_element_type=f32)
                   + jnp.dot(o_ref[...], w1b_ref[...], preferred_element_type=f32)
                   + b1_ref[...])
    out_all = _leaky(jnp.dot(h_all, w2_ref[...], preferred_element_type=f32)
                     + b2_ref[...])

    ui_b = jnp.broadcast_to(ui_t[:, None, :], (BBLK, N, R * H)).reshape(TB, R * H)
    ou = out_all * ui_b

    scores_t = jax.lax.dot_general(
        gt_ref[...], ou, (((1,), (1,)), ((), ())),
        preferred_element_type=f32)

    s_row = s_ref[0]
    oh = jax.lax.broadcasted_iota(jnp.int32, (R, TB), 0) == s_row
    out_ref[...] = jnp.sum(jnp.where(oh, scores_t, 0.0), axis=0,
                           keepdims=True)[None]


@jax.jit
def _run(u_emb, i_emb, a2, o2, s3,
         wui1a, wui1b, bui1, Wui2, bui2, w1a, w1b, b1_all, w2_bd, b2_all, gt):
    full = lambda shape: pl.BlockSpec(shape, lambda b: (0,) * len(shape))
    out3 = pl.pallas_call(
        _block_kernel,
        grid=(GRID,),
        in_specs=[
            pl.BlockSpec((BBLK, D), lambda b: (b, 0)),
            pl.BlockSpec((BBLK, D), lambda b: (b, 0)),
            pl.BlockSpec((TB, D), lambda b: (b, 0)),
            pl.BlockSpec((TB, D), lambda b: (b, 0)),
            pl.BlockSpec((1, 1, TB), lambda b: (b, 0, 0)),
            full((D, H)), full((D, H)), full((H,)), full((H, H)), full((H,)),
            full((D, R * H)), full((D, R * H)), full((R * H,)),
            full((R * H, R * H)), full((R * H,)), full((R, R * H)),
        ],
        out_specs=pl.BlockSpec((1, 1, TB), lambda b: (b, 0, 0)),
        out_shape=jax.ShapeDtypeStruct((GRID, 1, TB), jnp.float32),
        compiler_params=pltpu.CompilerParams(
            dimension_semantics=("parallel",),
        ),
    )(u_emb, i_emb, a2, o2, s3,
      wui1a, wui1b, bui1, Wui2, bui2, w1a, w1b, b1_all, w2_bd, b2_all, gt)
    return out3.reshape(B, N)


def kernel(u_emb, i_emb, a_emb, o_emb, s,
           Wui1, bui1, Wui2, bui2, Wao1, bao1, Wao2, bao2):
    w1_all = jnp.transpose(Wao1, (1, 0, 2)).reshape(2 * D, R * H)
    b1_all = bao1.reshape(R * H)
    eye = jnp.eye(R, dtype=Wao2.dtype)
    w2_bd = jnp.einsum('rkj,rq->rkqj', Wao2, eye).reshape(R * H, R * H)
    b2_all = bao2.reshape(R * H)
    gt = jnp.repeat(jnp.eye(R, dtype=jnp.float32), H, axis=0).T
    a2 = a_emb.reshape(B * N, D)
    o2 = o_emb.reshape(B * N, D)
    s3 = s.reshape(GRID, 1, TB)
    return _run(u_emb, i_emb, a2, o2, s3,
                Wui1[:D], Wui1[D:], bui1, Wui2, bui2,
                w1_all[:D], w1_all[D:], b1_all, w2_bd, b2_all, gt)
